# Initial kernel scaffold; baseline (speedup 1.0000x reference)
import jax
import jax.numpy as jnp
from jax.experimental import pallas as pl


def kernel(x, edge_index, W, b, W_hi, b_hi, W_lo, b_lo):
    raise NotImplementedError("write your pallas kernel here")


# speedup vs baseline: 21.9808x; 21.9808x over previous
import functools

import jax
import jax.numpy as jnp
from jax import lax
from jax.experimental import pallas as pl
from jax.experimental.pallas import tpu as pltpu
from jax.experimental.pallas import tpu_sc as plsc

N = 10000
E = 320000
D = 128
A_DIM = 64

NC = 2
NS = 16
L = 16
NW = NC * NS

NPAD = 10240
RPT = NPAD // NS

K = 80
ET = E // NW
NCHUNK = ET // K
CHPAD = 128
ROWS2D = NW * CHPAD

_mesh = plsc.VectorSubcoreMesh(core_axis_name="c", subcore_axis_name="s")


@functools.partial(
    pl.kernel,
    mesh=_mesh,
    out_type=jax.ShapeDtypeStruct((NC * NPAD, D), jnp.float32),
    scratch_types=[
        pltpu.VMEM((CHPAD, K), jnp.int32),
        pltpu.VMEM((K, D), jnp.float32),
        pltpu.VMEM((64, D), jnp.float32),
        pltpu.VMEM_SHARED((NPAD, D), jnp.float32),
    ],
)
def _deg_kernel(col_hbm, ones16_hbm, degp_hbm, col_v, ones_v, stage, deg_sh):
    c = lax.axis_index("c")
    s = lax.axis_index("s")
    wid = c * NS + s
    zero = jnp.zeros((L,), dtype=jnp.float32)
    for r in range(L):
        for q in range(D // L):
            stage[r, pl.ds(q * L, L)] = zero
    pltpu.sync_copy(col_hbm.at[pl.ds(wid * CHPAD, CHPAD)], col_v)
    for i in range(RPT // L):
        pltpu.sync_copy(stage.at[pl.ds(0, L)],
                        deg_sh.at[pl.ds(s * RPT + i * L, L)])
    for i in range(K // L):
        pltpu.sync_copy(ones16_hbm, ones_v.at[pl.ds(i * L, L)])
    plsc.subcore_barrier()
    for j in range(NCHUNK):
        pltpu.sync_copy(ones_v, deg_sh.at[col_v.at[j]], add=True)
    plsc.subcore_barrier()
    for i in range(RPT // 64):
        off = s * RPT + i * 64
        pltpu.sync_copy(deg_sh.at[pl.ds(off, 64)], stage)
        pltpu.sync_copy(stage, degp_hbm.at[pl.ds(c * NPAD + off, 64)])


def _h2_body(degp_ref, x_ref, w_ref, h2_ref):
    degsum = degp_ref[0:NPAD, :] + degp_ref[NPAD:, :]
    deg = jnp.sum(degsum, axis=1, keepdims=True) * (1.0 / D) + 1.0
    dis = lax.rsqrt(deg)
    h = jnp.dot(x_ref[...], w_ref[...], preferred_element_type=jnp.float32)
    h2_ref[...] = h * dis[:N]


_h2_call = pl.pallas_call(
    _h2_body,
    out_shape=jax.ShapeDtypeStruct((N, D), jnp.float32),
)


@functools.partial(
    pl.kernel,
    mesh=_mesh,
    out_type=jax.ShapeDtypeStruct((NC * NPAD, D), jnp.float32),
    scratch_types=[
        pltpu.VMEM((CHPAD, K), jnp.int32),
        pltpu.VMEM((CHPAD, K), jnp.int32),
        pltpu.VMEM((K, D), jnp.float32),
        pltpu.SemaphoreType.DMA,
        pltpu.VMEM_SHARED((NPAD, D), jnp.float32),
    ],
)
def _acc_kernel(h2_hbm, row_hbm, col_hbm, accp_hbm,
                row_v, col_v, rows_v, sem, acc_sh):
    c = lax.axis_index("c")
    s = lax.axis_index("s")
    wid = c * NS + s
    zero = jnp.zeros((L,), dtype=jnp.float32)
    for r in range(L):
        for q in range(D // L):
            rows_v[r, pl.ds(q * L, L)] = zero
    pltpu.sync_copy(row_hbm.at[pl.ds(wid * CHPAD, CHPAD)], row_v)
    pltpu.sync_copy(col_hbm.at[pl.ds(wid * CHPAD, CHPAD)], col_v)
    for i in range(RPT // L):
        pltpu.sync_copy(rows_v.at[pl.ds(0, L)],
                        acc_sh.at[pl.ds(s * RPT + i * L, L)])
    plsc.subcore_barrier()
    for j in range(NCHUNK):
        pltpu.async_copy(h2_hbm.at[row_v.at[j]], rows_v, sem).wait()
        pltpu.sync_copy(rows_v, acc_sh.at[col_v.at[j]], add=True)
    plsc.subcore_barrier()
    for i in range(RPT // 64):
        off = s * RPT + i * 64
        pltpu.sync_copy(acc_sh.at[pl.ds(off, 64)], rows_v.at[pl.ds(0, 64)])
        pltpu.sync_copy(rows_v.at[pl.ds(0, 64)],
                        accp_hbm.at[pl.ds(c * NPAD + off, 64)])


def _out_body(degp_ref, accp_ref, h2_ref, b_ref, whi_ref, bhi_ref,
              wlo_ref, blo_ref, hi_ref, lo_ref):
    degsum = degp_ref[0:NPAD, :] + degp_ref[NPAD:, :]
    deg = jnp.sum(degsum, axis=1, keepdims=True) * (1.0 / D) + 1.0
    dis = lax.rsqrt(deg)[:N]
    acc = accp_ref[0:N, :] + accp_ref[NPAD:NPAD + N, :] + h2_ref[...]
    out = acc * dis + b_ref[...]
    hi_ref[...] = jnp.dot(out, whi_ref[...],
                          preferred_element_type=jnp.float32) + bhi_ref[...]
    lo_ref[...] = jnp.dot(out, wlo_ref[...],
                          preferred_element_type=jnp.float32) + blo_ref[...]


_out_call = pl.pallas_call(
    _out_body,
    out_shape=(
        jax.ShapeDtypeStruct((N, A_DIM), jnp.float32),
        jax.ShapeDtypeStruct((N, A_DIM), jnp.float32),
    ),
)


def _edge_layout(e):
    e3 = e.reshape(NW, NCHUNK, K)
    e3 = jnp.pad(e3, ((0, 0), (0, CHPAD - NCHUNK), (0, 0)))
    return e3.reshape(ROWS2D, K)


def kernel(x, edge_index, W, b, W_hi, b_hi, W_lo, b_lo):
    row2d = _edge_layout(edge_index[0])
    col2d = _edge_layout(edge_index[1])
    ones16 = jnp.ones((L, D), dtype=jnp.float32)
    degp = _deg_kernel(col2d, ones16)
    h2 = _h2_call(degp, x, W)
    accp = _acc_kernel(h2, row2d, col2d)
    hi, lo = _out_call(degp, accp, h2, b.reshape(1, D),
                       W_hi, b_hi.reshape(1, A_DIM),
                       W_lo, b_lo.reshape(1, A_DIM))
    return (hi, lo)


# speedup vs baseline: 29.2940x; 1.3327x over previous
import functools

import jax
import jax.numpy as jnp
from jax import lax
from jax.experimental import pallas as pl
from jax.experimental.pallas import tpu as pltpu
from jax.experimental.pallas import tpu_sc as plsc

N = 10000
E = 320000
D = 128
A_DIM = 64

NC = 2
NS = 16
L = 16
NW = NC * NS

NPAD = 10240
RPT = NPAD // NS

K = 80
ET = E // NW
NCHUNK = ET // K
CHPAD = 128
ROWS2D = NW * CHPAD

_mesh = plsc.VectorSubcoreMesh(core_axis_name="c", subcore_axis_name="s")


@functools.partial(
    pl.kernel,
    mesh=_mesh,
    out_type=jax.ShapeDtypeStruct((NC * NPAD, D), jnp.float32),
    scratch_types=[
        pltpu.VMEM((CHPAD, K), jnp.int32),
        pltpu.VMEM((K, D), jnp.float32),
        pltpu.VMEM((64, D), jnp.float32),
        pltpu.SemaphoreType.DMA,
        pltpu.VMEM_SHARED((NPAD, D), jnp.float32),
    ],
)
def _deg_kernel(col_hbm, ones16_hbm, degp_hbm, col_v, ones_v, stage, sem,
                deg_sh):
    c = lax.axis_index("c")
    s = lax.axis_index("s")
    wid = c * NS + s
    zero = jnp.zeros((L,), dtype=jnp.float32)
    for r in range(L):
        for q in range(D // L):
            stage[r, pl.ds(q * L, L)] = zero
    pltpu.sync_copy(col_hbm.at[pl.ds(wid * CHPAD, CHPAD)], col_v)
    for i in range(RPT // L):
        pltpu.sync_copy(stage.at[pl.ds(0, L)],
                        deg_sh.at[pl.ds(s * RPT + i * L, L)])
    for i in range(K // L):
        pltpu.sync_copy(ones16_hbm, ones_v.at[pl.ds(i * L, L)])
    plsc.subcore_barrier()
    DEPTH = 8
    cps = []
    for j in range(NCHUNK):
        cps.append(pltpu.async_copy(ones_v, deg_sh.at[col_v.at[j]], sem,
                                    add=True))
        if len(cps) >= DEPTH:
            cps.pop(0).wait()
    for cp in cps:
        cp.wait()
    plsc.subcore_barrier()
    for i in range(RPT // 64):
        off = s * RPT + i * 64
        pltpu.sync_copy(deg_sh.at[pl.ds(off, 64)], stage)
        pltpu.sync_copy(stage, degp_hbm.at[pl.ds(c * NPAD + off, 64)])


def _h1_body(x_ref, w_ref, h_ref):
    h_ref[...] = jnp.dot(x_ref[...], w_ref[...],
                         preferred_element_type=jnp.float32)


_h1_call = pl.pallas_call(
    _h1_body,
    out_shape=jax.ShapeDtypeStruct((N, D), jnp.float32),
)


def _h2_body(degp_ref, h_ref, h2_ref):
    degsum = degp_ref[0:NPAD, :] + degp_ref[NPAD:, :]
    deg = jnp.sum(degsum, axis=1, keepdims=True) * (1.0 / D) + 1.0
    dis = lax.rsqrt(deg)
    h2_ref[...] = h_ref[...] * dis[:N]


_h2_call = pl.pallas_call(
    _h2_body,
    out_shape=jax.ShapeDtypeStruct((N, D), jnp.float32),
)


@functools.partial(
    pl.kernel,
    mesh=_mesh,
    out_type=jax.ShapeDtypeStruct((NC * NPAD, D), jnp.float32),
    scratch_types=[
        pltpu.VMEM((64, K), jnp.int32),
        pltpu.VMEM((64, K), jnp.int32),
        pltpu.VMEM((K, D), jnp.float32),
        pltpu.VMEM((K, D), jnp.float32),
        pltpu.SemaphoreType.DMA,
        pltpu.SemaphoreType.DMA,
        pltpu.VMEM_SHARED((NPAD, D), jnp.float32),
    ],
)
def _acc_kernel(h2_hbm, row_hbm, col_hbm, accp_hbm,
                row_v, col_v, rows_a, rows_b, sem_a, sem_b, acc_sh):
    c = lax.axis_index("c")
    s = lax.axis_index("s")
    wid = c * NS + s
    zero = jnp.zeros((L,), dtype=jnp.float32)
    for r in range(L):
        for q in range(D // L):
            rows_a[r, pl.ds(q * L, L)] = zero
    for i in range(RPT // L):
        pltpu.sync_copy(rows_a.at[pl.ds(0, L)],
                        acc_sh.at[pl.ds(s * RPT + i * L, L)])
    plsc.subcore_barrier()
    bufs = (rows_a, rows_b)
    sems = (sem_a, sem_b)
    for ph in range(2):
        nj = 64 if ph == 0 else NCHUNK - 64
        pltpu.sync_copy(row_hbm.at[pl.ds(wid * CHPAD + ph * 64, 64)], row_v)
        pltpu.sync_copy(col_hbm.at[pl.ds(wid * CHPAD + ph * 64, 64)], col_v)
        pend = pltpu.async_copy(h2_hbm.at[row_v.at[0]], bufs[0], sems[0])
        for j in range(nj):
            cur = j % 2
            nxt_pend = None
            if j + 1 < nj:
                nxt_pend = pltpu.async_copy(h2_hbm.at[row_v.at[j + 1]],
                                            bufs[1 - cur], sems[1 - cur])
            pend.wait()
            pltpu.sync_copy(bufs[cur], acc_sh.at[col_v.at[j]], add=True)
            pend = nxt_pend
    plsc.subcore_barrier()
    for i in range(RPT // 64):
        off = s * RPT + i * 64
        pltpu.sync_copy(acc_sh.at[pl.ds(off, 64)], rows_a.at[pl.ds(0, 64)])
        pltpu.sync_copy(rows_a.at[pl.ds(0, 64)],
                        accp_hbm.at[pl.ds(c * NPAD + off, 64)])


def _out_body(degp_ref, accp_ref, h2_ref, b_ref, whi_ref, bhi_ref,
              wlo_ref, blo_ref, hi_ref, lo_ref):
    degsum = degp_ref[0:NPAD, :] + degp_ref[NPAD:, :]
    deg = jnp.sum(degsum, axis=1, keepdims=True) * (1.0 / D) + 1.0
    dis = lax.rsqrt(deg)[:N]
    acc = accp_ref[0:N, :] + accp_ref[NPAD:NPAD + N, :] + h2_ref[...]
    out = acc * dis + b_ref[...]
    hi_ref[...] = jnp.dot(out, whi_ref[...],
                          preferred_element_type=jnp.float32) + bhi_ref[...]
    lo_ref[...] = jnp.dot(out, wlo_ref[...],
                          preferred_element_type=jnp.float32) + blo_ref[...]


_out_call = pl.pallas_call(
    _out_body,
    out_shape=(
        jax.ShapeDtypeStruct((N, A_DIM), jnp.float32),
        jax.ShapeDtypeStruct((N, A_DIM), jnp.float32),
    ),
)


def _edge_layout(e):
    e3 = e.reshape(NW, NCHUNK, K)
    e3 = jnp.pad(e3, ((0, 0), (0, CHPAD - NCHUNK), (0, 0)))
    return e3.reshape(ROWS2D, K)


def kernel(x, edge_index, W, b, W_hi, b_hi, W_lo, b_lo):
    row2d = _edge_layout(edge_index[0])
    col2d = _edge_layout(edge_index[1])
    ones16 = jnp.ones((L, D), dtype=jnp.float32)
    degp = _deg_kernel(col2d, ones16)
    h = _h1_call(x, W)
    h2 = _h2_call(degp, h)
    accp = _acc_kernel(h2, row2d, col2d)
    hi, lo = _out_call(degp, accp, h2, b.reshape(1, D),
                       W_hi, b_hi.reshape(1, A_DIM),
                       W_lo, b_lo.reshape(1, A_DIM))
    return (hi, lo)
